# baseline (speedup 1.0000x reference)
import functools
import math

import jax
import jax.numpy as jnp
from jax import lax
from jax.experimental import pallas as pl
from jax.experimental.pallas import tpu as pltpu
from jax.experimental.pallas import tpu_sc as plsc

VOCAB = 1000000
D = 64
MAX_LEN = 200
BATCH = 4096
SEQ = 200

NC = 2
NS = 16
NW = NC * NS
BB = BATCH // NW
L = 16
NB = 2


def _positional_encoding() -> jnp.ndarray:
    w = jnp.exp(-jnp.arange(0, D, 2, dtype=jnp.float32) * math.log(10000.0) / D)
    p = jnp.arange(0, MAX_LEN, dtype=jnp.float32).reshape(MAX_LEN, 1)
    pe = jnp.zeros((MAX_LEN, D), dtype=jnp.float32)
    pe = pe.at[:, 0::2].set(jnp.sin(p * w))
    pe = pe.at[:, 1::2].set(jnp.cos(p * w))
    return pe


def _body(xT_hbm, tab_hbm, pe_hbm, out_hbm,
          idx_v, i20, i21, r0, r1, o0, o1, pe_v,
          gs0, gs1, ws0, ws1):
    wid = lax.axis_index("s") * NC + lax.axis_index("c")
    idx2 = (i20, i21)
    rows = (r0, r1)
    outs = (o0, o1)
    gsem = (gs0, gs1)
    wsem = (ws0, ws1)

    pltpu.sync_copy(xT_hbm.at[:, pl.ds(wid * BB, BB)], idx_v)
    pltpu.sync_copy(pe_hbm, pe_v)

    def start_gather(s, b):
        for g in range(BB // L):
            sl = pl.ds(g * L, L)
            idx2[b][sl] = lax.shift_right_logical(idx_v[s, sl], 1)
        pltpu.async_copy(tab_hbm.at[idx2[b]], rows[b], gsem[b])

    def wait_gather(b):
        pltpu.make_async_copy(tab_hbm.at[idx2[b]], rows[b], gsem[b]).wait()

    def start_write(s, b):
        pltpu.async_copy(
            outs[b], out_hbm.at[s, :, pl.ds(wid * BB, BB)], wsem[b])

    def wait_write(b):
        pltpu.make_async_copy(
            outs[b], out_hbm.at[0, :, pl.ds(wid * BB, BB)], wsem[b]).wait()

    for b in range(NB):
        start_gather(b, b)

    @pl.loop(0, SEQ, step=NB)
    def outer(s0):
        for b in range(NB):
            s = s0 + b
            wait_gather(b)

            @pl.when(s0 > 0)
            def _():
                wait_write(b)

            offs = []
            for g in range(BB // L):
                offs.append((idx_v[s, pl.ds(g * L, L)] & 1) * D)
            iotas = [
                lax.broadcasted_iota(jnp.int32, (L,), 0) + g * L
                for g in range(BB // L)
            ]

            for dg in range(D // L):
                pe_vec = pe_v[s, pl.ds(dg * L, L)]

                @pl.loop(0, L)
                def _dd(dd):
                    d = dg * L + dd
                    pe_b = pe_vec.at[jnp.full((L,), dd, jnp.int32)].get(
                        mode="promise_in_bounds")
                    for g in range(BB // L):
                        val = plsc.load_gather(rows[b], [iotas[g], offs[g] + d])
                        outs[b][d, pl.ds(g * L, L)] = val * 8.0 + pe_b

            @pl.when(s < SEQ - NB)
            def _():
                start_gather(s + NB, b)

            start_write(s, b)

    for b in range(NB):
        wait_write(b)


def kernel(x, table):
    xT = x.T
    tab = table.reshape(VOCAB // 2, 2 * D)
    pe = _positional_encoding()[:SEQ]

    mesh = plsc.VectorSubcoreMesh(core_axis_name="c", subcore_axis_name="s")
    k = functools.partial(
        pl.kernel,
        mesh=mesh,
        out_type=jax.ShapeDtypeStruct((SEQ, D, BATCH), jnp.float32),
        scratch_types=[
            pltpu.VMEM((SEQ, BB), jnp.int32),
            pltpu.VMEM((BB,), jnp.int32),
            pltpu.VMEM((BB,), jnp.int32),
            pltpu.VMEM((BB, 2 * D), jnp.float32),
            pltpu.VMEM((BB, 2 * D), jnp.float32),
            pltpu.VMEM((D, BB), jnp.float32),
            pltpu.VMEM((D, BB), jnp.float32),
            pltpu.VMEM((SEQ, D), jnp.float32),
            pltpu.SemaphoreType.DMA,
            pltpu.SemaphoreType.DMA,
            pltpu.SemaphoreType.DMA,
            pltpu.SemaphoreType.DMA,
        ],
        compiler_params=pltpu.CompilerParams(
            use_tc_tiling_on_sc=True, needs_layout_passes=False),
    )(_body)
    out_t = k(xT, tab, pe)
    return out_t.transpose(2, 0, 1)


# speedup vs baseline: 1.0218x; 1.0218x over previous
import functools
import math

import jax
import jax.numpy as jnp
from jax import lax
from jax.experimental import pallas as pl
from jax.experimental.pallas import tpu as pltpu
from jax.experimental.pallas import tpu_sc as plsc

VOCAB = 1000000
D = 64
MAX_LEN = 200
BATCH = 4096
SEQ = 200

NC = 2
NS = 16
NW = NC * NS
BB = BATCH // NW
L = 16
NB = 2


def _positional_encoding() -> jnp.ndarray:
    w = jnp.exp(-jnp.arange(0, D, 2, dtype=jnp.float32) * math.log(10000.0) / D)
    p = jnp.arange(0, MAX_LEN, dtype=jnp.float32).reshape(MAX_LEN, 1)
    pe = jnp.zeros((MAX_LEN, D), dtype=jnp.float32)
    pe = pe.at[:, 0::2].set(jnp.sin(p * w))
    pe = pe.at[:, 1::2].set(jnp.cos(p * w))
    return pe


OBW = BB + 1


def _body(xT_hbm, tab_hbm, pe_hbm, out_hbm,
          idx_v, i20, i21, of0, of1, r0, r1, o0, o1, pe_v,
          gs0, gs1, ws0, ws1):
    wid = lax.axis_index("s") * NC + lax.axis_index("c")
    idx2 = (i20, i21)
    offb = (of0, of1)
    rows = (r0, r1)
    outs = (o0, o1)
    gsem = (gs0, gs1)
    wsem = (ws0, ws1)

    pltpu.sync_copy(xT_hbm.at[:, pl.ds(wid * BB, BB)], idx_v)
    pltpu.sync_copy(pe_hbm, pe_v)

    iota = lax.broadcasted_iota(jnp.int32, (L,), 0)
    dgiota = [iota + dg * L for dg in range(D // L)]

    def start_gather(s, b):
        for g in range(BB // L):
            sl = pl.ds(g * L, L)
            v = idx_v[s, sl]
            idx2[b][sl] = lax.shift_right_logical(v, 1)
            offb[b][sl] = (v & 1) * D
        pltpu.async_copy(tab_hbm.at[idx2[b]], rows[b], gsem[b])

    def wait_gather(b):
        pltpu.make_async_copy(tab_hbm.at[idx2[b]], rows[b], gsem[b]).wait()

    def start_write(s, b):
        pltpu.async_copy(
            outs[b].at[:, pl.ds(0, BB)],
            out_hbm.at[s, :, pl.ds(wid * BB, BB)], wsem[b])

    def wait_write(b):
        pltpu.make_async_copy(
            outs[b].at[:, pl.ds(0, BB)],
            out_hbm.at[0, :, pl.ds(wid * BB, BB)], wsem[b]).wait()

    for b in range(NB):
        start_gather(b, b)

    @pl.loop(0, SEQ, step=NB)
    def outer(s0):
        for b in range(NB):
            s = s0 + b
            wait_gather(b)

            @pl.when(s0 > 0)
            def _():
                wait_write(b)

            pes = [pe_v[s, pl.ds(dg * L, L)] for dg in range(D // L)]

            @pl.loop(0, BB, unroll=4)
            def _bb(bb):
                sp = jnp.full((L,), bb, jnp.int32)
                off = plsc.load_gather(offb[b], [sp])
                for dg in range(D // L):
                    val = plsc.load_gather(rows[b], [sp, off + dgiota[dg]])
                    o = val * 8.0 + pes[dg]
                    plsc.store_scatter(outs[b], [dgiota[dg], sp], o)

            @pl.when(s < SEQ - NB)
            def _():
                start_gather(s + NB, b)

            start_write(s, b)

    for b in range(NB):
        wait_write(b)


def kernel(x, table):
    xT = x.T
    tab = table.reshape(VOCAB // 2, 2 * D)
    pe = _positional_encoding()[:SEQ]

    mesh = plsc.VectorSubcoreMesh(core_axis_name="c", subcore_axis_name="s")
    k = functools.partial(
        pl.kernel,
        mesh=mesh,
        out_type=jax.ShapeDtypeStruct((SEQ, D, BATCH), jnp.float32),
        scratch_types=[
            pltpu.VMEM((SEQ, BB), jnp.int32),
            pltpu.VMEM((BB,), jnp.int32),
            pltpu.VMEM((BB,), jnp.int32),
            pltpu.VMEM((BB,), jnp.int32),
            pltpu.VMEM((BB,), jnp.int32),
            pltpu.VMEM((BB, 2 * D), jnp.float32),
            pltpu.VMEM((BB, 2 * D), jnp.float32),
            pltpu.VMEM((D, OBW), jnp.float32),
            pltpu.VMEM((D, OBW), jnp.float32),
            pltpu.VMEM((SEQ, D), jnp.float32),
            pltpu.SemaphoreType.DMA,
            pltpu.SemaphoreType.DMA,
            pltpu.SemaphoreType.DMA,
            pltpu.SemaphoreType.DMA,
        ],
        compiler_params=pltpu.CompilerParams(
            use_tc_tiling_on_sc=True, needs_layout_passes=False),
    )(_body)
    out_t = k(xT, tab, pe)
    return out_t.transpose(2, 0, 1)
